# Initial kernel scaffold; baseline (speedup 1.0000x reference)
import jax
import jax.numpy as jnp
from jax.experimental import pallas as pl


def kernel(fake_images, real_images):
    raise NotImplementedError("write your pallas kernel here")


# speedup vs baseline: 10.6630x; 10.6630x over previous
import functools

import jax
import jax.numpy as jnp
from jax import lax
from jax.experimental import pallas as pl
from jax.experimental.pallas import tpu as pltpu
from jax.experimental.pallas import tpu_sc as plsc

_WS = 8
_BINS = 256


def _tc_body(x_ref, oi_ref, om_ref, od_ref):
    x = x_ref[...]
    nw = x.shape[2] // _WS

    def su(a):
        return jnp.concatenate([a[:, 1:, :], jnp.zeros_like(a[:, :1, :])], axis=1)

    def sd(a):
        return jnp.concatenate([jnp.zeros_like(a[:, :1, :]), a[:, :-1, :]], axis=1)

    def sl(a):
        return jnp.concatenate([a[:, :, 1:], jnp.zeros_like(a[:, :, :1])], axis=2)

    def sr(a):
        return jnp.concatenate([jnp.zeros_like(a[:, :, :1]), a[:, :, :-1]], axis=2)

    t = sd(x) + 2.0 * x + su(x)
    ex = sl(t) - sr(t)
    s = sr(x) + 2.0 * x + sl(x)
    ey = su(s) - sd(s)
    mag = jnp.sqrt(ex * ex + ey * ey)
    ang = jnp.arctan2(ey, ex)

    r_io = lax.broadcasted_iota(jnp.int32, x.shape, 1)
    c_io = lax.broadcasted_iota(jnp.int32, x.shape, 2)
    wbase = (((r_io >> 3) * nw) + (c_io >> 3)) * _BINS

    def bidx(v):
        mn = jnp.min(v)
        mx = jnp.max(v)
        width = (mx - mn) / _BINS
        safe_w = jnp.where(width == 0.0, 1.0, width)
        iv = jnp.floor((v - mn) / safe_w).astype(jnp.int32)
        return wbase + jnp.clip(iv, 0, _BINS - 1)

    oi_ref[...] = bidx(x)
    om_ref[...] = bidx(mag)
    od_ref[...] = bidx(ang)


def _make_sc(nchunks, cw, nh, nw, batch):
    info = plsc.get_sparse_core_info()
    nworkers = info.num_cores * info.num_subcores
    assert nh % nworkers == 0
    bands_per_w = nh // nworkers
    acc_n = bands_per_w * nw * _BINS
    lanes = 16

    @functools.partial(
        pl.kernel,
        out_type=jax.ShapeDtypeStruct((nworkers, 2 * lanes), jnp.float32),
        mesh=plsc.VectorSubcoreMesh(core_axis_name="c", subcore_axis_name="s"),
        compiler_params=pltpu.CompilerParams(needs_layout_passes=False),
        scratch_types=[
            pltpu.VMEM((acc_n,), jnp.float32),
            pltpu.VMEM((acc_n,), jnp.float32),
            pltpu.VMEM((cw,), jnp.int32),
            pltpu.VMEM((2 * lanes,), jnp.float32),
        ],
    )
    def sc(i_f, i_r, i_fm, i_fd, i_rm, i_rd, out, hist, edge, stg, ovec):
        cid = lax.axis_index("c")
        sid = lax.axis_index("s")
        wid = sid * info.num_cores + cid
        zero16 = jnp.zeros((lanes,), jnp.float32)

        def zbody(i, carry):
            hist[pl.ds(i * lanes, lanes)] = zero16
            edge[pl.ds(i * lanes, lanes)] = zero16
            return carry

        lax.fori_loop(0, acc_n // lanes, zbody, 0)

        base = wid * acc_n
        fields = (
            (i_f, 1.0, hist), (i_r, -1.0, hist),
            (i_fm, 1.0, edge), (i_fd, 1.0, edge),
            (i_rm, -1.0, edge), (i_rd, -1.0, edge),
        )
        for ref, sign, acc in fields:
            sgn = jnp.full((lanes,), sign, jnp.float32)
            for bl in range(bands_per_w):
                for b in range(batch):
                    band = wid * bands_per_w + bl
                    chunk = b * nh + band
                    pltpu.sync_copy(ref.at[chunk], stg)

                    def sbody(i, carry):
                        v = stg[pl.ds(i * lanes, lanes)]
                        plsc.addupdate_scatter(acc, [v - base], sgn)
                        return carry

                    lax.fori_loop(0, cw // lanes, sbody, 0)

        def abody(i, carry):
            h, e = carry
            return (h + jnp.abs(hist[pl.ds(i * lanes, lanes)]),
                    e + jnp.abs(edge[pl.ds(i * lanes, lanes)]))

        hs, es = lax.fori_loop(0, acc_n // lanes, abody, (zero16, zero16))
        ovec[pl.ds(0, lanes)] = hs
        ovec[pl.ds(lanes, lanes)] = es
        pltpu.sync_copy(ovec, out.at[wid])

    return sc


def kernel(fake_images, real_images):
    B, C, H, W = fake_images.shape
    nh, nw = H // _WS, W // _WS
    batch = B * C
    f3 = fake_images.reshape(batch, H, W)
    r3 = real_images.reshape(batch, H, W)

    out_sds = [jax.ShapeDtypeStruct((batch, H, W), jnp.int32)] * 3
    tc = pl.pallas_call(_tc_body, out_shape=out_sds)
    fi, fm, fd = tc(f3)
    ri, rm, rd = tc(r3)

    cw = _WS * W
    nchunks = batch * nh
    shape2 = (nchunks, cw)
    sc = _make_sc(nchunks, cw, nh, nw, batch)
    parts = sc(fi.reshape(shape2), ri.reshape(shape2), fm.reshape(shape2),
               fd.reshape(shape2), rm.reshape(shape2), rd.reshape(shape2))
    return jnp.sum(parts) / (nh * nw * _BINS)


# speedup vs baseline: 15.2443x; 1.4296x over previous
import functools

import jax
import jax.numpy as jnp
from jax import lax
from jax.experimental import pallas as pl
from jax.experimental.pallas import tpu as pltpu
from jax.experimental.pallas import tpu_sc as plsc

_WS = 8
_BINS = 256
_BANDS_PER_W = 2


def _tc_body(x_ref, oi_ref, om_ref, od_ref):
    x = x_ref[...]
    nw = x.shape[2] // _WS

    def su(a):
        return jnp.concatenate([a[:, 1:, :], jnp.zeros_like(a[:, :1, :])], axis=1)

    def sd(a):
        return jnp.concatenate([jnp.zeros_like(a[:, :1, :]), a[:, :-1, :]], axis=1)

    def sl(a):
        return jnp.concatenate([a[:, :, 1:], jnp.zeros_like(a[:, :, :1])], axis=2)

    def sr(a):
        return jnp.concatenate([jnp.zeros_like(a[:, :, :1]), a[:, :, :-1]], axis=2)

    t = sd(x) + 2.0 * x + su(x)
    ex = sl(t) - sr(t)
    s = sr(x) + 2.0 * x + sl(x)
    ey = su(s) - sd(s)
    mag = jnp.sqrt(ex * ex + ey * ey)
    ang = jnp.arctan2(ey, ex)

    r_io = lax.broadcasted_iota(jnp.int32, x.shape, 1)
    c_io = lax.broadcasted_iota(jnp.int32, x.shape, 2)
    wbase = ((((r_io >> 3) % _BANDS_PER_W) * nw) + (c_io >> 3)) * _BINS

    def bidx(v):
        mn = jnp.min(v)
        mx = jnp.max(v)
        width = (mx - mn) / _BINS
        safe_w = jnp.where(width == 0.0, 1.0, width)
        iv = jnp.floor((v - mn) / safe_w).astype(jnp.int32)
        return wbase + jnp.clip(iv, 0, _BINS - 1)

    oi_ref[...] = bidx(x)
    om_ref[...] = bidx(mag)
    od_ref[...] = bidx(ang)


def _make_sc(nchunks, cw, nh, nw, batch):
    info = plsc.get_sparse_core_info()
    nworkers = info.num_cores * info.num_subcores
    assert nh % nworkers == 0
    bands_per_w = nh // nworkers
    assert bands_per_w == _BANDS_PER_W
    acc_n = bands_per_w * nw * _BINS
    lanes = 16
    unroll = 8
    step = lanes * unroll

    @functools.partial(
        pl.kernel,
        out_type=jax.ShapeDtypeStruct((nworkers, 2 * lanes), jnp.float32),
        mesh=plsc.VectorSubcoreMesh(core_axis_name="c", subcore_axis_name="s"),
        compiler_params=pltpu.CompilerParams(needs_layout_passes=False),
        scratch_types=[
            pltpu.VMEM((acc_n,), jnp.float32),
            pltpu.VMEM((acc_n,), jnp.float32),
            pltpu.VMEM((cw,), jnp.int32),
            pltpu.VMEM((cw,), jnp.int32),
            pltpu.VMEM((2 * lanes,), jnp.float32),
            pltpu.SemaphoreType.DMA,
            pltpu.SemaphoreType.DMA,
        ],
    )
    def sc(i_f, i_r, i_fm, i_fd, i_rm, i_rd, out, hist, edge, stg0, stg1,
           ovec, sem0, sem1):
        cid = lax.axis_index("c")
        sid = lax.axis_index("s")
        wid = sid * info.num_cores + cid
        zero16 = jnp.zeros((lanes,), jnp.float32)

        def zbody(i, carry):
            b0 = i * step
            for j in range(unroll):
                hist[pl.ds(b0 + j * lanes, lanes)] = zero16
                edge[pl.ds(b0 + j * lanes, lanes)] = zero16
            return carry

        lax.fori_loop(0, acc_n // step, zbody, 0)

        fields = (
            (i_f, 1.0, hist), (i_r, -1.0, hist),
            (i_fm, 1.0, edge), (i_fd, 1.0, edge),
            (i_rm, -1.0, edge), (i_rd, -1.0, edge),
        )
        items = []
        for ref, sign, acc in fields:
            for bl in range(bands_per_w):
                for b in range(batch):
                    chunk = b * nh + wid * bands_per_w + bl
                    items.append((ref, sign, acc, chunk))

        stg = (stg0, stg1)
        sem = (sem0, sem1)
        copies = [None, None]
        copies[0] = pltpu.async_copy(items[0][0].at[items[0][3]], stg[0], sem[0])
        for k, (ref, sign, acc, chunk) in enumerate(items):
            cur, nxt = k % 2, (k + 1) % 2
            if k + 1 < len(items):
                nref, _, _, nchunk = items[k + 1]
                copies[nxt] = pltpu.async_copy(nref.at[nchunk], stg[nxt], sem[nxt])
            copies[cur].wait()
            sgn = jnp.full((lanes,), sign, jnp.float32)
            buf = stg[cur]

            def sbody(i, carry):
                b0 = i * step
                for j in range(unroll):
                    v = buf[pl.ds(b0 + j * lanes, lanes)]
                    plsc.addupdate_scatter(acc, [v], sgn)
                return carry

            lax.fori_loop(0, cw // step, sbody, 0)

        def abody(i, carry):
            parts = list(carry)
            b0 = i * step
            for j in range(unroll):
                parts[j % 4] = parts[j % 4] + jnp.abs(hist[pl.ds(b0 + j * lanes, lanes)])
                parts[4 + j % 4] = parts[4 + j % 4] + jnp.abs(edge[pl.ds(b0 + j * lanes, lanes)])
            return tuple(parts)

        parts = lax.fori_loop(0, acc_n // step, abody, (zero16,) * 8)
        ovec[pl.ds(0, lanes)] = (parts[0] + parts[1]) + (parts[2] + parts[3])
        ovec[pl.ds(lanes, lanes)] = (parts[4] + parts[5]) + (parts[6] + parts[7])
        pltpu.sync_copy(ovec, out.at[wid])

    return sc


def kernel(fake_images, real_images):
    B, C, H, W = fake_images.shape
    nh, nw = H // _WS, W // _WS
    batch = B * C
    f3 = fake_images.reshape(batch, H, W)
    r3 = real_images.reshape(batch, H, W)

    out_sds = [jax.ShapeDtypeStruct((batch, H, W), jnp.int32)] * 3
    tc = pl.pallas_call(_tc_body, out_shape=out_sds)
    fi, fm, fd = tc(f3)
    ri, rm, rd = tc(r3)

    cw = _WS * W
    nchunks = batch * nh
    shape2 = (nchunks, cw)
    sc = _make_sc(nchunks, cw, nh, nw, batch)
    parts = sc(fi.reshape(shape2), ri.reshape(shape2), fm.reshape(shape2),
               fd.reshape(shape2), rm.reshape(shape2), rd.reshape(shape2))
    return jnp.sum(parts) / (nh * nw * _BINS)


# speedup vs baseline: 15.2732x; 1.0019x over previous
import functools

import jax
import jax.numpy as jnp
from jax import lax
from jax.experimental import pallas as pl
from jax.experimental.pallas import tpu as pltpu
from jax.experimental.pallas import tpu_sc as plsc

_WS = 8
_BINS = 256
_BANDS_PER_W = 2


def _tc_body(f_ref, r_ref, fi_ref, fm_ref, fd_ref, ri_ref, rm_ref, rd_ref):
    for x_ref, oi_ref, om_ref, od_ref in ((f_ref, fi_ref, fm_ref, fd_ref),
                                          (r_ref, ri_ref, rm_ref, rd_ref)):
        _tc_one(x_ref, oi_ref, om_ref, od_ref)


def _tc_one(x_ref, oi_ref, om_ref, od_ref):
    x = x_ref[...]
    nw = x.shape[2] // _WS

    def su(a):
        return jnp.concatenate([a[:, 1:, :], jnp.zeros_like(a[:, :1, :])], axis=1)

    def sd(a):
        return jnp.concatenate([jnp.zeros_like(a[:, :1, :]), a[:, :-1, :]], axis=1)

    def sl(a):
        return jnp.concatenate([a[:, :, 1:], jnp.zeros_like(a[:, :, :1])], axis=2)

    def sr(a):
        return jnp.concatenate([jnp.zeros_like(a[:, :, :1]), a[:, :, :-1]], axis=2)

    t = sd(x) + 2.0 * x + su(x)
    ex = sl(t) - sr(t)
    s = sr(x) + 2.0 * x + sl(x)
    ey = su(s) - sd(s)
    mag = jnp.sqrt(ex * ex + ey * ey)
    ang = jnp.arctan2(ey, ex)

    r_io = lax.broadcasted_iota(jnp.int32, x.shape, 1)
    c_io = lax.broadcasted_iota(jnp.int32, x.shape, 2)
    wbase = ((((r_io >> 3) % _BANDS_PER_W) * nw) + (c_io >> 3)) * _BINS

    def bidx(v):
        mn = jnp.min(v)
        mx = jnp.max(v)
        width = (mx - mn) / _BINS
        safe_w = jnp.where(width == 0.0, 1.0, width)
        iv = jnp.floor((v - mn) / safe_w).astype(jnp.int32)
        return wbase + jnp.clip(iv, 0, _BINS - 1)

    oi_ref[...] = bidx(x)
    om_ref[...] = bidx(mag)
    od_ref[...] = bidx(ang)


def _make_sc(nchunks, cw, nh, nw, batch):
    info = plsc.get_sparse_core_info()
    nworkers = info.num_cores * info.num_subcores
    assert nh % nworkers == 0
    bands_per_w = nh // nworkers
    assert bands_per_w == _BANDS_PER_W
    acc_n = bands_per_w * nw * _BINS
    lanes = 16
    unroll = 16
    step = lanes * unroll

    @functools.partial(
        pl.kernel,
        out_type=jax.ShapeDtypeStruct((nworkers, 2 * lanes), jnp.float32),
        mesh=plsc.VectorSubcoreMesh(core_axis_name="c", subcore_axis_name="s"),
        compiler_params=pltpu.CompilerParams(needs_layout_passes=False),
        scratch_types=[
            pltpu.VMEM((acc_n,), jnp.float32),
            pltpu.VMEM((acc_n,), jnp.float32),
            pltpu.VMEM((cw,), jnp.int32),
            pltpu.VMEM((cw,), jnp.int32),
            pltpu.VMEM((2 * lanes,), jnp.float32),
            pltpu.SemaphoreType.DMA,
            pltpu.SemaphoreType.DMA,
        ],
    )
    def sc(i_f, i_r, i_fm, i_fd, i_rm, i_rd, out, hist, edge, stg0, stg1,
           ovec, sem0, sem1):
        cid = lax.axis_index("c")
        sid = lax.axis_index("s")
        wid = sid * info.num_cores + cid
        zero16 = jnp.zeros((lanes,), jnp.float32)

        def zbody(i, carry):
            b0 = i * step
            for j in range(unroll):
                hist[pl.ds(b0 + j * lanes, lanes)] = zero16
                edge[pl.ds(b0 + j * lanes, lanes)] = zero16
            return carry

        lax.fori_loop(0, acc_n // step, zbody, 0)

        fields = (
            (i_f, 1.0, hist), (i_r, -1.0, hist),
            (i_fm, 1.0, edge), (i_fd, 1.0, edge),
            (i_rm, -1.0, edge), (i_rd, -1.0, edge),
        )
        items = []
        for ref, sign, acc in fields:
            for bl in range(bands_per_w):
                for b in range(batch):
                    chunk = b * nh + wid * bands_per_w + bl
                    items.append((ref, sign, acc, chunk))

        stg = (stg0, stg1)
        sem = (sem0, sem1)
        copies = [None, None]
        copies[0] = pltpu.async_copy(items[0][0].at[items[0][3]], stg[0], sem[0])
        for k, (ref, sign, acc, chunk) in enumerate(items):
            cur, nxt = k % 2, (k + 1) % 2
            if k + 1 < len(items):
                nref, _, _, nchunk = items[k + 1]
                copies[nxt] = pltpu.async_copy(nref.at[nchunk], stg[nxt], sem[nxt])
            copies[cur].wait()
            sgn = jnp.full((lanes,), sign, jnp.float32)
            buf = stg[cur]

            def sbody(i, carry):
                b0 = i * step
                for j in range(unroll):
                    v = buf[pl.ds(b0 + j * lanes, lanes)]
                    plsc.addupdate_scatter(acc, [v], sgn)
                return carry

            lax.fori_loop(0, cw // step, sbody, 0)

        def abody(i, carry):
            parts = list(carry)
            b0 = i * step
            for j in range(unroll):
                parts[j % 4] = parts[j % 4] + jnp.abs(hist[pl.ds(b0 + j * lanes, lanes)])
                parts[4 + j % 4] = parts[4 + j % 4] + jnp.abs(edge[pl.ds(b0 + j * lanes, lanes)])
            return tuple(parts)

        parts = lax.fori_loop(0, acc_n // step, abody, (zero16,) * 8)
        ovec[pl.ds(0, lanes)] = (parts[0] + parts[1]) + (parts[2] + parts[3])
        ovec[pl.ds(lanes, lanes)] = (parts[4] + parts[5]) + (parts[6] + parts[7])
        pltpu.sync_copy(ovec, out.at[wid])

    return sc


def kernel(fake_images, real_images):
    B, C, H, W = fake_images.shape
    nh, nw = H // _WS, W // _WS
    batch = B * C
    f3 = fake_images.reshape(batch, H, W)
    r3 = real_images.reshape(batch, H, W)

    out_sds = [jax.ShapeDtypeStruct((batch, H, W), jnp.int32)] * 6
    tc = pl.pallas_call(
        _tc_body, out_shape=out_sds,
        compiler_params=pltpu.CompilerParams(vmem_limit_bytes=120 * 1024 * 1024))
    fi, fm, fd, ri, rm, rd = tc(f3, r3)

    cw = _WS * W
    nchunks = batch * nh
    shape2 = (nchunks, cw)
    sc = _make_sc(nchunks, cw, nh, nw, batch)
    parts = sc(fi.reshape(shape2), ri.reshape(shape2), fm.reshape(shape2),
               fd.reshape(shape2), rm.reshape(shape2), rd.reshape(shape2))
    return jnp.sum(parts) / (nh * nw * _BINS)


# speedup vs baseline: 20.1498x; 1.3193x over previous
import functools

import jax
import jax.numpy as jnp
from jax import lax
from jax.experimental import pallas as pl
from jax.experimental.pallas import tpu as pltpu
from jax.experimental.pallas import tpu_sc as plsc

_WS = 8
_BINS = 256
_BANDS_PER_W = 2


def _tc_body(f_ref, r_ref, fi_ref, fm_ref, fd_ref, ri_ref, rm_ref, rd_ref):
    for x_ref, oi_ref, om_ref, od_ref in ((f_ref, fi_ref, fm_ref, fd_ref),
                                          (r_ref, ri_ref, rm_ref, rd_ref)):
        _tc_one(x_ref, oi_ref, om_ref, od_ref)


def _tc_one(x_ref, oi_ref, om_ref, od_ref):
    x = x_ref[...]
    nw = x.shape[2] // _WS

    def su(a):
        return jnp.concatenate([a[:, 1:, :], jnp.zeros_like(a[:, :1, :])], axis=1)

    def sd(a):
        return jnp.concatenate([jnp.zeros_like(a[:, :1, :]), a[:, :-1, :]], axis=1)

    def sl(a):
        return jnp.concatenate([a[:, :, 1:], jnp.zeros_like(a[:, :, :1])], axis=2)

    def sr(a):
        return jnp.concatenate([jnp.zeros_like(a[:, :, :1]), a[:, :, :-1]], axis=2)

    t = sd(x) + 2.0 * x + su(x)
    ex = sl(t) - sr(t)
    s = sr(x) + 2.0 * x + sl(x)
    ey = su(s) - sd(s)
    mag = jnp.sqrt(ex * ex + ey * ey)
    ang = jnp.arctan2(ey, ex)

    r_io = lax.broadcasted_iota(jnp.int32, x.shape, 1)
    c_io = lax.broadcasted_iota(jnp.int32, x.shape, 2)
    wbase = ((((r_io >> 3) % _BANDS_PER_W) * nw) + (c_io >> 3)) * _BINS

    def bidx(v):
        mn = jnp.min(v)
        mx = jnp.max(v)
        width = (mx - mn) / _BINS
        inv_w = jnp.where(width == 0.0, 1.0, 1.0 / width)
        iv = jnp.floor((v - mn) * inv_w).astype(jnp.int32)
        return wbase + jnp.clip(iv, 0, _BINS - 1)

    oi_ref[...] = bidx(x)
    om_ref[...] = bidx(mag)
    od_ref[...] = bidx(ang)


def _make_sc(nchunks, cw, nh, nw, batch):
    info = plsc.get_sparse_core_info()
    nworkers = info.num_cores * info.num_subcores
    assert nh % nworkers == 0
    bands_per_w = nh // nworkers
    assert bands_per_w == _BANDS_PER_W
    acc_n = bands_per_w * nw * _BINS
    lanes = 16
    unroll = 16
    step = lanes * unroll

    @functools.partial(
        pl.kernel,
        out_type=jax.ShapeDtypeStruct((nworkers, 2 * lanes), jnp.float32),
        mesh=plsc.VectorSubcoreMesh(core_axis_name="c", subcore_axis_name="s"),
        compiler_params=pltpu.CompilerParams(needs_layout_passes=False),
        scratch_types=[
            pltpu.VMEM((acc_n,), jnp.float32),
            pltpu.VMEM((acc_n,), jnp.float32),
            pltpu.VMEM((cw,), jnp.int32),
            pltpu.VMEM((cw,), jnp.int32),
            pltpu.VMEM((2 * lanes,), jnp.float32),
            pltpu.SemaphoreType.DMA,
            pltpu.SemaphoreType.DMA,
        ],
    )
    def sc(i_f, i_r, i_fm, i_fd, i_rm, i_rd, out, hist, edge, stg0, stg1,
           ovec, sem0, sem1):
        cid = lax.axis_index("c")
        sid = lax.axis_index("s")
        wid = sid * info.num_cores + cid
        zero16 = jnp.zeros((lanes,), jnp.float32)

        def zbody(i, carry):
            b0 = i * step
            for j in range(unroll):
                hist[pl.ds(b0 + j * lanes, lanes)] = zero16
                edge[pl.ds(b0 + j * lanes, lanes)] = zero16
            return carry

        lax.fori_loop(0, acc_n // step, zbody, 0)

        fields = (
            (i_f, 1.0, hist), (i_r, -1.0, hist),
            (i_fm, 1.0, edge), (i_fd, 1.0, edge),
            (i_rm, -1.0, edge), (i_rd, -1.0, edge),
        )
        items = []
        for ref, sign, acc in fields:
            for bl in range(bands_per_w):
                for b in range(batch):
                    chunk = b * nh + wid * bands_per_w + bl
                    items.append((ref, sign, acc, chunk))

        stg = (stg0, stg1)
        sem = (sem0, sem1)
        copies = [None, None]
        copies[0] = pltpu.async_copy(items[0][0].at[items[0][3]], stg[0], sem[0])
        for k, (ref, sign, acc, chunk) in enumerate(items):
            cur, nxt = k % 2, (k + 1) % 2
            if k + 1 < len(items):
                nref, _, _, nchunk = items[k + 1]
                copies[nxt] = pltpu.async_copy(nref.at[nchunk], stg[nxt], sem[nxt])
            copies[cur].wait()
            sgn = jnp.full((lanes,), sign, jnp.float32)
            buf = stg[cur]

            def sbody(i, carry):
                b0 = i * step
                vs = [buf[pl.ds(b0 + j * lanes, lanes)] for j in range(unroll)]
                for v in vs:
                    plsc.addupdate_scatter(acc, [v], sgn)
                return carry

            lax.fori_loop(0, cw // step, sbody, 0)

        aunroll = 8
        astep = lanes * aunroll

        def abody(i, carry):
            parts = list(carry)
            b0 = i * astep
            hv = [hist[pl.ds(b0 + j * lanes, lanes)] for j in range(aunroll)]
            ev = [edge[pl.ds(b0 + j * lanes, lanes)] for j in range(aunroll)]
            for j in range(aunroll):
                parts[j % 4] = parts[j % 4] + jnp.abs(hv[j])
                parts[4 + j % 4] = parts[4 + j % 4] + jnp.abs(ev[j])
            return tuple(parts)

        parts = lax.fori_loop(0, acc_n // astep, abody, (zero16,) * 8)
        ovec[pl.ds(0, lanes)] = (parts[0] + parts[1]) + (parts[2] + parts[3])
        ovec[pl.ds(lanes, lanes)] = (parts[4] + parts[5]) + (parts[6] + parts[7])
        pltpu.sync_copy(ovec, out.at[wid])

    return sc


def kernel(fake_images, real_images):
    B, C, H, W = fake_images.shape
    nh, nw = H // _WS, W // _WS
    batch = B * C
    f3 = fake_images.reshape(batch, H, W)
    r3 = real_images.reshape(batch, H, W)

    out_sds = [jax.ShapeDtypeStruct((batch, H, W), jnp.int32)] * 6
    tc = pl.pallas_call(
        _tc_body, out_shape=out_sds,
        compiler_params=pltpu.CompilerParams(vmem_limit_bytes=120 * 1024 * 1024))
    fi, fm, fd, ri, rm, rd = tc(f3, r3)

    cw = _WS * W
    nchunks = batch * nh
    shape2 = (nchunks, cw)
    sc = _make_sc(nchunks, cw, nh, nw, batch)
    parts = sc(fi.reshape(shape2), ri.reshape(shape2), fm.reshape(shape2),
               fd.reshape(shape2), rm.reshape(shape2), rd.reshape(shape2))
    return jnp.sum(parts) / (nh * nw * _BINS)


# speedup vs baseline: 20.6114x; 1.0229x over previous
import functools

import jax
import jax.numpy as jnp
from jax import lax
from jax.experimental import pallas as pl
from jax.experimental.pallas import tpu as pltpu
from jax.experimental.pallas import tpu_sc as plsc

_ATAN_Q = (0.999999958195304, -0.33332302827701, 0.19973681363348264,
           -0.14040138890720566, 0.09967923617773113, -0.060219127974898964,
           0.024756780680341817, -0.004831168384696569)
_PI = 3.14159265358979323846


def _atan2(y, x):
    ax = jnp.abs(x)
    ay = jnp.abs(y)
    swap = ay > ax
    num = jnp.where(swap, ax, ay)
    den = jnp.where(swap, ay, ax)
    den = jnp.where(den == 0.0, 1.0, den)
    z = num / den
    t = z * z
    p = jnp.float32(_ATAN_Q[-1])
    for c in _ATAN_Q[-2::-1]:
        p = p * t + jnp.float32(c)
    a = z * p
    a = jnp.where(swap, jnp.float32(_PI / 2) - a, a)
    a = jnp.where(x < 0.0, jnp.float32(_PI) - a, a)
    return jnp.where(y < 0.0, -a, a)


_WS = 8
_BINS = 256
_BANDS_PER_W = 2


def _tc_body(f_ref, r_ref, fi_ref, fm_ref, fd_ref, ri_ref, rm_ref, rd_ref):
    for x_ref, oi_ref, om_ref, od_ref in ((f_ref, fi_ref, fm_ref, fd_ref),
                                          (r_ref, ri_ref, rm_ref, rd_ref)):
        _tc_one(x_ref, oi_ref, om_ref, od_ref)


def _tc_one(x_ref, oi_ref, om_ref, od_ref):
    x = x_ref[...]
    nw = x.shape[2] // _WS

    def su(a):
        return jnp.concatenate([a[:, 1:, :], jnp.zeros_like(a[:, :1, :])], axis=1)

    def sd(a):
        return jnp.concatenate([jnp.zeros_like(a[:, :1, :]), a[:, :-1, :]], axis=1)

    def sl(a):
        return jnp.concatenate([a[:, :, 1:], jnp.zeros_like(a[:, :, :1])], axis=2)

    def sr(a):
        return jnp.concatenate([jnp.zeros_like(a[:, :, :1]), a[:, :, :-1]], axis=2)

    t = sd(x) + 2.0 * x + su(x)
    ex = sl(t) - sr(t)
    s = sr(x) + 2.0 * x + sl(x)
    ey = su(s) - sd(s)
    mag = jnp.sqrt(ex * ex + ey * ey)
    ang = _atan2(ey, ex)

    r_io = lax.broadcasted_iota(jnp.int32, x.shape, 1)
    c_io = lax.broadcasted_iota(jnp.int32, x.shape, 2)
    wbase = ((((r_io >> 3) % _BANDS_PER_W) * nw) + (c_io >> 3)) * _BINS

    def bidx(v):
        mn = jnp.min(v)
        mx = jnp.max(v)
        width = (mx - mn) / _BINS
        inv_w = jnp.where(width == 0.0, 1.0, 1.0 / width)
        iv = jnp.floor((v - mn) * inv_w).astype(jnp.int32)
        return wbase + jnp.clip(iv, 0, _BINS - 1)

    oi_ref[...] = bidx(x)
    om_ref[...] = bidx(mag)
    od_ref[...] = bidx(ang)


def _make_sc(nchunks, cw, nh, nw, batch):
    info = plsc.get_sparse_core_info()
    nworkers = info.num_cores * info.num_subcores
    assert nh % nworkers == 0
    bands_per_w = nh // nworkers
    assert bands_per_w == _BANDS_PER_W
    acc_n = bands_per_w * nw * _BINS
    lanes = 16
    unroll = 16
    step = lanes * unroll

    @functools.partial(
        pl.kernel,
        out_type=jax.ShapeDtypeStruct((nworkers, 2 * lanes), jnp.float32),
        mesh=plsc.VectorSubcoreMesh(core_axis_name="c", subcore_axis_name="s"),
        compiler_params=pltpu.CompilerParams(needs_layout_passes=False),
        scratch_types=[
            pltpu.VMEM((acc_n,), jnp.float32),
            pltpu.VMEM((acc_n,), jnp.float32),
            pltpu.VMEM((cw,), jnp.int32),
            pltpu.VMEM((cw,), jnp.int32),
            pltpu.VMEM((2 * lanes,), jnp.float32),
            pltpu.SemaphoreType.DMA,
            pltpu.SemaphoreType.DMA,
        ],
    )
    def sc(i_f, i_r, i_fm, i_fd, i_rm, i_rd, out, hist, edge, stg0, stg1,
           ovec, sem0, sem1):
        cid = lax.axis_index("c")
        sid = lax.axis_index("s")
        wid = sid * info.num_cores + cid
        zero16 = jnp.zeros((lanes,), jnp.float32)

        def zbody(i, carry):
            b0 = i * step
            for j in range(unroll):
                hist[pl.ds(b0 + j * lanes, lanes)] = zero16
                edge[pl.ds(b0 + j * lanes, lanes)] = zero16
            return carry

        lax.fori_loop(0, acc_n // step, zbody, 0)

        fields = (
            (i_f, 1.0, hist), (i_r, -1.0, hist),
            (i_fm, 1.0, edge), (i_fd, 1.0, edge),
            (i_rm, -1.0, edge), (i_rd, -1.0, edge),
        )
        items = []
        for ref, sign, acc in fields:
            for bl in range(bands_per_w):
                for b in range(batch):
                    chunk = b * nh + wid * bands_per_w + bl
                    items.append((ref, sign, acc, chunk))

        stg = (stg0, stg1)
        sem = (sem0, sem1)
        copies = [None, None]
        copies[0] = pltpu.async_copy(items[0][0].at[items[0][3]], stg[0], sem[0])
        for k, (ref, sign, acc, chunk) in enumerate(items):
            cur, nxt = k % 2, (k + 1) % 2
            if k + 1 < len(items):
                nref, _, _, nchunk = items[k + 1]
                copies[nxt] = pltpu.async_copy(nref.at[nchunk], stg[nxt], sem[nxt])
            copies[cur].wait()
            sgn = jnp.full((lanes,), sign, jnp.float32)
            buf = stg[cur]

            def sbody(i, carry):
                b0 = i * step
                vs = [buf[pl.ds(b0 + j * lanes, lanes)] for j in range(unroll)]
                for v in vs:
                    plsc.addupdate_scatter(acc, [v], sgn)
                return carry

            lax.fori_loop(0, cw // step, sbody, 0)

        aunroll = 8
        astep = lanes * aunroll

        def abody(i, carry):
            parts = list(carry)
            b0 = i * astep
            hv = [hist[pl.ds(b0 + j * lanes, lanes)] for j in range(aunroll)]
            ev = [edge[pl.ds(b0 + j * lanes, lanes)] for j in range(aunroll)]
            for j in range(aunroll):
                parts[j % 4] = parts[j % 4] + jnp.abs(hv[j])
                parts[4 + j % 4] = parts[4 + j % 4] + jnp.abs(ev[j])
            return tuple(parts)

        parts = lax.fori_loop(0, acc_n // astep, abody, (zero16,) * 8)
        ovec[pl.ds(0, lanes)] = (parts[0] + parts[1]) + (parts[2] + parts[3])
        ovec[pl.ds(lanes, lanes)] = (parts[4] + parts[5]) + (parts[6] + parts[7])
        pltpu.sync_copy(ovec, out.at[wid])

    return sc


def kernel(fake_images, real_images):
    B, C, H, W = fake_images.shape
    nh, nw = H // _WS, W // _WS
    batch = B * C
    f3 = fake_images.reshape(batch, H, W)
    r3 = real_images.reshape(batch, H, W)

    out_sds = [jax.ShapeDtypeStruct((batch, H, W), jnp.int32)] * 6
    tc = pl.pallas_call(
        _tc_body, out_shape=out_sds,
        compiler_params=pltpu.CompilerParams(vmem_limit_bytes=120 * 1024 * 1024))
    fi, fm, fd, ri, rm, rd = tc(f3, r3)

    cw = _WS * W
    nchunks = batch * nh
    shape2 = (nchunks, cw)
    sc = _make_sc(nchunks, cw, nh, nw, batch)
    parts = sc(fi.reshape(shape2), ri.reshape(shape2), fm.reshape(shape2),
               fd.reshape(shape2), rm.reshape(shape2), rd.reshape(shape2))
    return jnp.sum(parts) / (nh * nw * _BINS)
